# baseline (speedup 1.0000x reference)
import jax
import jax.numpy as jnp
from jax import lax
from jax.experimental import pallas as pl
from jax.experimental.pallas import tpu as pltpu, tpu_sc as plsc

B = 4 * 8192
D = 512
B_SC = B // 2
NC = 2
NS = 16
NW = NC * NS
BPW = B_SC // NW
CHUNK = 64
NCHUNK = BPW // CHUNK
NBUF = 3
RPW = 32
REP_ROWS = NW * RPW * 2
RB = 512
NBLK = (B - B_SC) // RB
BLK_OFF = B_SC // RB


def _sc_body(x_hbm, table_hbm, out_hbm, rep_hbm,
             idx_v, tbl_v, rows_v, gsem, osem):
    wid = lax.axis_index("s") * NC + lax.axis_index("c")
    pltpu.sync_copy(x_hbm.at[wid], idx_v)
    pltpu.sync_copy(table_hbm, tbl_v)

    reps = []
    for r in range(RPW):
        c = pltpu.make_async_copy(
            tbl_v, rep_hbm.at[pl.ds((wid * RPW + r) * 2, 2)], osem)
        c.start()
        reps.append(c)

    off0 = 2 * (wid * RPW) + 2 * lax.iota(jnp.int32, 16)
    for c16 in range(NCHUNK):
        for g in range(CHUNK // 16):
            sl = pl.ds(g * 16, 16)
            idx_v[c16, sl] = idx_v[c16, sl] + (off0 + (g % 2) * 32)

    for c in reps:
        c.wait()

    base = wid * BPW
    gathers = [None] * NCHUNK
    outs = [None] * NCHUNK
    for j in range(min(NBUF, NCHUNK)):
        gathers[j] = pltpu.make_async_copy(
            rep_hbm.at[idx_v.at[j]], rows_v.at[j % NBUF], gsem)
        gathers[j].start()
    for j in range(NCHUNK):
        b = j % NBUF
        gathers[j].wait()
        outs[j] = pltpu.make_async_copy(
            rows_v.at[b], out_hbm.at[pl.ds(base + j * CHUNK, CHUNK)], osem)
        outs[j].start()
        nj = j + NBUF
        if nj < NCHUNK:
            outs[j].wait()
            gathers[nj] = pltpu.make_async_copy(
                rep_hbm.at[idx_v.at[nj]], rows_v.at[b], gsem)
            gathers[nj].start()
    for j in range(max(0, NCHUNK - NBUF), NCHUNK):
        outs[j].wait()


def _tc_body(x_ref, tbl_ref, alias_ref, out_ref):
    del alias_ref
    xb = x_ref[...]
    out_ref[...] = jnp.where(xb == 0, tbl_ref[0:1, :], tbl_ref[1:2, :])


def kernel(x, table):
    x_flat = x.reshape(B).astype(jnp.int32)
    xf = x_flat[:B_SC].reshape(NW, NCHUNK, CHUNK)
    half, _ = pl.kernel(
        _sc_body,
        out_type=[
            jax.ShapeDtypeStruct((B, D), jnp.float32),
            jax.ShapeDtypeStruct((REP_ROWS, D), jnp.float32),
        ],
        mesh=plsc.VectorSubcoreMesh(core_axis_name="c", subcore_axis_name="s"),
        scratch_types=[
            pltpu.VMEM((NCHUNK, CHUNK), jnp.int32),
            pltpu.VMEM((2, D), jnp.float32),
            pltpu.VMEM((NBUF, CHUNK, D), jnp.float32),
            pltpu.SemaphoreType.DMA,
            pltpu.SemaphoreType.DMA,
        ],
    )(xf, table)

    x2 = x_flat[B_SC:].reshape(B - B_SC, 1)
    out = pl.pallas_call(
        _tc_body,
        grid=(NBLK,),
        in_specs=[
            pl.BlockSpec((RB, 1), lambda j: (j, 0)),
            pl.BlockSpec((2, D), lambda j: (0, 0)),
            pl.BlockSpec((RB, D), lambda j: (BLK_OFF + j, 0)),
        ],
        out_specs=pl.BlockSpec((RB, D), lambda j: (BLK_OFF + j, 0)),
        out_shape=jax.ShapeDtypeStruct((B, D), jnp.float32),
        input_output_aliases={2: 0},
    )(x2, table, half)
    return out.reshape(x.shape[0], x.shape[1], D)


# speedup vs baseline: 1.1597x; 1.1597x over previous
import jax
import jax.numpy as jnp
from jax import lax
from jax.experimental import pallas as pl
from jax.experimental.pallas import tpu as pltpu, tpu_sc as plsc

B = 4 * 8192
D = 512
NC = 2
NS = 16
NW = NC * NS
BPW = B // NW
CHUNK = 64
NCHUNK = BPW // CHUNK
NBUF = 3
RPW = 64
REP_ROWS = NW * RPW * 2


def _sc_body(x_hbm, table_hbm, out_hbm, rep_hbm,
             idx_v, tbl_v, rows_v, gsem, osem):
    wid = lax.axis_index("s") * NC + lax.axis_index("c")
    pltpu.sync_copy(x_hbm.at[wid], idx_v)
    pltpu.sync_copy(table_hbm, tbl_v)

    reps = []
    for r in range(RPW):
        c = pltpu.make_async_copy(
            tbl_v, rep_hbm.at[pl.ds((wid * RPW + r) * 2, 2)], osem)
        c.start()
        reps.append(c)

    off0 = 2 * (wid * RPW) + 2 * lax.iota(jnp.int32, 16)
    for c16 in range(NCHUNK):
        for g in range(CHUNK // 16):
            sl = pl.ds(g * 16, 16)
            idx_v[c16, sl] = idx_v[c16, sl] + (off0 + (g % 4) * 32)

    for c in reps:
        c.wait()

    base = wid * BPW
    gathers = [None] * NCHUNK
    outs = [None] * NCHUNK
    for j in range(min(NBUF, NCHUNK)):
        gathers[j] = pltpu.make_async_copy(
            rep_hbm.at[idx_v.at[j]], rows_v.at[j % NBUF], gsem)
        gathers[j].start()
    for j in range(NCHUNK):
        b = j % NBUF
        gathers[j].wait()
        outs[j] = pltpu.make_async_copy(
            rows_v.at[b], out_hbm.at[pl.ds(base + j * CHUNK, CHUNK)], osem)
        outs[j].start()
        nj = j + NBUF
        if nj < NCHUNK:
            outs[j].wait()
            gathers[nj] = pltpu.make_async_copy(
                rep_hbm.at[idx_v.at[nj]], rows_v.at[b], gsem)
            gathers[nj].start()
    for j in range(max(0, NCHUNK - NBUF), NCHUNK):
        outs[j].wait()


def kernel(x, table):
    xf = x.reshape(NW, NCHUNK, CHUNK).astype(jnp.int32)
    out, _ = pl.kernel(
        _sc_body,
        out_type=[
            jax.ShapeDtypeStruct((B, D), jnp.float32),
            jax.ShapeDtypeStruct((REP_ROWS, D), jnp.float32),
        ],
        mesh=plsc.VectorSubcoreMesh(core_axis_name="c", subcore_axis_name="s"),
        scratch_types=[
            pltpu.VMEM((NCHUNK, CHUNK), jnp.int32),
            pltpu.VMEM((2, D), jnp.float32),
            pltpu.VMEM((NBUF, CHUNK, D), jnp.float32),
            pltpu.SemaphoreType.DMA,
            pltpu.SemaphoreType.DMA,
        ],
    )(xf, table)
    return out.reshape(x.shape[0], x.shape[1], D)
